# baseline (speedup 1.0000x reference)
import functools

import jax
import jax.numpy as jnp
from jax import lax
from jax.experimental import pallas as pl
from jax.experimental.pallas import tpu as pltpu
from jax.experimental.pallas import tpu_sc as plsc

N_NODES = 50000
NF = 64
HALF = 32
B = 512
N_RBF = 51
RBF_PAD = 64
GAMMA = 10.0
STEP = 0.1

E_PAD = 802816
EDGES_PER_TILE32 = E_PAD // 32
EDGES_PER_TILE16 = E_PAD // 16
MSG_BLK = 128
MSG_NBLK = EDGES_PER_TILE16 // MSG_BLK
DIST_CHUNK = 3136
DIST_NCHUNK = EDGES_PER_TILE32 // DIST_CHUNK
ACC_ROWS = 50048
DUMP_ROW = N_NODES
ROWS_PER_TILE = ACC_ROWS // 16

R_BLK = 1000
N_GRID = N_NODES // R_BLK
C_BLK = 2048
C_GRID = E_PAD // C_BLK


def _ssp(v):
    return jax.nn.softplus(v) - 0.6931471805599453


def _dist_body(px, py, pz, src, dst, d2_out, posbuf, sidx, didx, d2buf, dummy):
    del dummy
    wid = lax.axis_index("s") * 2 + lax.axis_index("c")
    base = wid * EDGES_PER_TILE32

    def _zero(i, _):
        d2buf[pl.ds(i * 16, 16)] = jnp.zeros((16,), jnp.float32)
        return 0
    lax.fori_loop(0, EDGES_PER_TILE32 // 16, _zero, 0)

    for coord in (px, py, pz):
        pltpu.sync_copy(coord, posbuf)

        def _chunk(c, _):
            off = base + c * DIST_CHUNK
            pltpu.sync_copy(src.at[pl.ds(off, DIST_CHUNK)], sidx)
            pltpu.sync_copy(dst.at[pl.ds(off, DIST_CHUNK)], didx)

            def _vec(i, _):
                s16 = sidx[pl.ds(i * 16, 16)]
                d16 = didx[pl.ds(i * 16, 16)]
                a = plsc.load_gather(posbuf, [s16])
                b = plsc.load_gather(posbuf, [d16])
                diff = a - b
                sl = pl.ds(c * DIST_CHUNK + i * 16, 16)
                d2buf[sl] = d2buf[sl] + diff * diff
                return 0
            lax.fori_loop(0, DIST_CHUNK // 16, _vec, 0)
            return 0
        lax.fori_loop(0, DIST_NCHUNK, _chunk, 0)

    pltpu.sync_copy(d2buf, d2_out.at[pl.ds(base, EDGES_PER_TILE32)])


GROUP = 14
MSG_NGRP = MSG_NBLK // GROUP


def _msg_body(src2, dst2, zeros, m0, m1, w0, w1, v0, v1,
              acc, sidx, didx, mb0, mb1, wb0, wb1, xb0, xb1,
              sg0, sg1, sw0, sw1, ss0, ss1):
    cid = lax.axis_index("c")
    tid = lax.axis_index("s")

    pltpu.sync_copy(zeros, acc.at[pl.ds(tid * ROWS_PER_TILE, ROWS_PER_TILE)])
    plsc.subcore_barrier()

    def _edges(m, w, vout):
        mb = (mb0, mb1)
        wb = (wb0, wb1)
        xb = (xb0, xb1)
        sg = (sg0, sg1)
        sw = (sw0, sw1)
        ss = (ss0, ss1)
        tbase128 = tid * (EDGES_PER_TILE16 // MSG_BLK)

        def _group(g, _):
            row0 = tbase128 + g * GROUP
            pltpu.sync_copy(src2.at[pl.ds(row0, GROUP)], sidx)
            pltpu.sync_copy(dst2.at[pl.ds(row0, GROUP)], didx)

            def _start(k):
                b = k & 1
                hm = pltpu.async_copy(m.at[sidx.at[k]], mb[b], sg[b])
                hw = pltpu.async_copy(
                    w.at[pl.ds((row0 + k) * MSG_BLK, MSG_BLK)], wb[b], sw[b])
                return hm, hw

            pend = {0: _start(0), 1: _start(1)}
            pend_s = {}
            for k in range(GROUP):
                b = k & 1
                hm, hw = pend.pop(k)
                hm.wait()
                hw.wait()
                if k - 2 in pend_s:
                    pend_s.pop(k - 2).wait()

                def _mul(q, _):
                    for r in range(4):
                        row = q * 4 + r
                        lo = pl.ds(0, 16)
                        hi = pl.ds(16, 16)
                        xb[b][row, lo] = mb[b][row, lo] * wb[b][row, lo]
                        xb[b][row, hi] = mb[b][row, hi] * wb[b][row, hi]
                    return 0
                lax.fori_loop(0, MSG_BLK // 4, _mul, 0)

                pend_s[k] = pltpu.async_copy(
                    xb[b], acc.at[didx.at[k]], ss[b], add=True)
                if k + 2 < GROUP:
                    pend[k + 2] = _start(k + 2)
            for k in (GROUP - 2, GROUP - 1):
                pend_s.pop(k).wait()
            return 0
        lax.fori_loop(0, MSG_NGRP, _group, 0)
        plsc.subcore_barrier()
        rows = pl.ds(tid * ROWS_PER_TILE, ROWS_PER_TILE)
        pltpu.sync_copy(acc.at[rows], vout.at[rows])

    @pl.when(cid == 0)
    def _():
        _edges(m0, w0, v0)

    @pl.when(cid == 1)
    def _():
        _edges(m1, w1, v1)


def _embed_body(z_ref, solv_ref, embz_ref, embs_ref, s1w, s1b, s2w, s2b,
                l1w, l1b, h_ref, m0_ref, m1_ref, s_ref):
    z = z_ref[0, 0, :].reshape(R_BLK, 1)
    zoh = (z == lax.broadcasted_iota(jnp.int32, (R_BLK, 100), 1)).astype(jnp.float32)
    h = jnp.dot(zoh, embz_ref[...], preferred_element_type=jnp.float32,
                precision=lax.Precision.HIGHEST)
    h_ref[...] = h
    m = jnp.dot(h, l1w[...], preferred_element_type=jnp.float32) + l1b[...]
    m0_ref[...] = m[:, :HALF]
    m1_ref[...] = m[:, HALF:]

    sv = solv_ref[0, 0, :].reshape(B, 1)
    soh = (sv == lax.broadcasted_iota(jnp.int32, (B, 4), 1)).astype(jnp.float32)
    es = jnp.dot(soh, embs_ref[...], preferred_element_type=jnp.float32,
                 precision=lax.Precision.HIGHEST)
    t = _ssp(jnp.dot(es, s1w[...], preferred_element_type=jnp.float32) + s1b[...])
    s_ref[...] = jnp.dot(t, s2w[...], preferred_element_type=jnp.float32) + s2b[...]


def _filter_body(d2_ref, mu_ref, f1a, b1a, f2a, b2a, f1b, b1b, f2b, b2b,
                 w10_ref, w11_ref, w20_ref, w21_ref):
    d = jnp.sqrt(d2_ref[...] + 1e-12)
    delta = d - mu_ref[...]
    rbf = jnp.exp(-GAMMA * delta * delta)
    t = _ssp(jnp.dot(rbf, f1a[...], preferred_element_type=jnp.float32) + b1a[...])
    w = jnp.dot(t, f2a[...], preferred_element_type=jnp.float32) + b2a[...]
    w10_ref[...] = w[:, :HALF]
    w11_ref[...] = w[:, HALF:]
    t = _ssp(jnp.dot(rbf, f1b[...], preferred_element_type=jnp.float32) + b1b[...])
    w = jnp.dot(t, f2b[...], preferred_element_type=jnp.float32) + b2b[...]
    w20_ref[...] = w[:, :HALF]
    w21_ref[...] = w[:, HALF:]


def _update1_body(h_ref, v0_ref, v1_ref, m1w, m1b, m2w, m2b, l2w, l2b,
                  h1_ref, m0_ref, m1_ref):
    v = jnp.concatenate([v0_ref[...], v1_ref[...]], axis=1)
    t = _ssp(jnp.dot(v, m1w[...], preferred_element_type=jnp.float32) + m1b[...])
    t = jnp.dot(t, m2w[...], preferred_element_type=jnp.float32) + m2b[...]
    h1 = h_ref[...] + t
    h1_ref[...] = h1
    m = jnp.dot(h1, l2w[...], preferred_element_type=jnp.float32) + l2b[...]
    m0_ref[...] = m[:, :HALF]
    m1_ref[...] = m[:, HALF:]


def _final_body(h_ref, v0_ref, v1_ref, batch_ref, s_ref,
                m1w, m1b, m2w, m2b, p1w, p1b, p2w, p2b,
                paw, pab, pbw, pbb, pcw, pcb,
                out_ref, acc, cnt):
    i = pl.program_id(0)
    v = jnp.concatenate([v0_ref[...], v1_ref[...]], axis=1)
    t = _ssp(jnp.dot(v, m1w[...], preferred_element_type=jnp.float32) + m1b[...])
    t = jnp.dot(t, m2w[...], preferred_element_type=jnp.float32) + m2b[...]
    h = h_ref[...] + t
    t = _ssp(jnp.dot(h, p1w[...], preferred_element_type=jnp.float32) + p1b[...])
    ht = jnp.dot(t, p2w[...], preferred_element_type=jnp.float32) + p2b[...]

    bv = batch_ref[0, 0, :].reshape(R_BLK, 1)
    oh = (bv == lax.broadcasted_iota(jnp.int32, (R_BLK, B), 1)).astype(jnp.float32)

    @pl.when(i == 0)
    def _():
        acc[...] = jnp.zeros((B, NF), jnp.float32)
        cnt[...] = jnp.zeros((B, 1), jnp.float32)

    dn = (((0,), (0,)), ((), ()))
    acc[...] = acc[...] + lax.dot_general(oh, ht, dn,
                                          preferred_element_type=jnp.float32,
                                          precision=lax.Precision.HIGHEST)
    cnt[...] = cnt[...] + lax.dot_general(
        oh, jnp.ones((R_BLK, 1), jnp.float32), dn,
        preferred_element_type=jnp.float32)

    @pl.when(i == N_GRID - 1)
    def _():
        gmean = acc[...] / jnp.maximum(cnt[...], 1.0)
        cat = jnp.concatenate([gmean, s_ref[...]], axis=1)
        o = _ssp(jnp.dot(cat, paw[...], preferred_element_type=jnp.float32) + pab[...])
        o = _ssp(jnp.dot(o, pbw[...], preferred_element_type=jnp.float32) + pbb[...])
        out_ref[...] = jnp.dot(o, pcw[...], preferred_element_type=jnp.float32) + pcb[...]


_MESH = plsc.VectorSubcoreMesh(core_axis_name="c", subcore_axis_name="s")
_SC_PARAMS = pltpu.CompilerParams(needs_layout_passes=False,
                                  use_tc_tiling_on_sc=False)

_dist_call = functools.partial(
    pl.kernel, _dist_body,
    out_type=jax.ShapeDtypeStruct((E_PAD,), jnp.float32),
    mesh=_MESH,
    scratch_types=[
        pltpu.VMEM((N_NODES,), jnp.float32),
        pltpu.VMEM((DIST_CHUNK,), jnp.int32),
        pltpu.VMEM((DIST_CHUNK,), jnp.int32),
        pltpu.VMEM((EDGES_PER_TILE32,), jnp.float32),
        pltpu.SemaphoreType.DMA,
    ],
    compiler_params=_SC_PARAMS,
)

_msg_call = functools.partial(
    pl.kernel, _msg_body,
    out_type=(jax.ShapeDtypeStruct((ACC_ROWS, HALF), jnp.float32),
              jax.ShapeDtypeStruct((ACC_ROWS, HALF), jnp.float32)),
    mesh=_MESH,
    scratch_types=[
        pltpu.VMEM_SHARED((ACC_ROWS, HALF), jnp.float32),
        pltpu.VMEM((GROUP, MSG_BLK), jnp.int32),
        pltpu.VMEM((GROUP, MSG_BLK), jnp.int32),
        pltpu.VMEM((MSG_BLK, HALF), jnp.float32),
        pltpu.VMEM((MSG_BLK, HALF), jnp.float32),
        pltpu.VMEM((MSG_BLK, HALF), jnp.float32),
        pltpu.VMEM((MSG_BLK, HALF), jnp.float32),
        pltpu.VMEM((MSG_BLK, HALF), jnp.float32),
        pltpu.VMEM((MSG_BLK, HALF), jnp.float32),
        pltpu.SemaphoreType.DMA,
        pltpu.SemaphoreType.DMA,
        pltpu.SemaphoreType.DMA,
        pltpu.SemaphoreType.DMA,
        pltpu.SemaphoreType.DMA,
        pltpu.SemaphoreType.DMA,
    ],
    compiler_params=_SC_PARAMS,
)


def _full(shape):
    return pl.BlockSpec(shape, lambda i: tuple(0 for _ in shape))


def kernel(x, pos, edge_index, Z, batch, solvent, nuc_index, params):
    del x, nuc_index
    f32 = jnp.float32
    p = params
    inter = p['interactions']

    src = edge_index[0]
    dst = edge_index[1]
    n_e = src.shape[0]
    pad = E_PAD - n_e
    src_p = jnp.concatenate([src, jnp.zeros((pad,), jnp.int32)])
    dst_p = jnp.concatenate([dst, jnp.zeros((pad,), jnp.int32)])
    dst_s = jnp.concatenate([dst, jnp.full((pad,), DUMP_ROW, jnp.int32)])

    px = pos[:, 0]
    py = pos[:, 1]
    pz = pos[:, 2]

    d2 = _dist_call()(px, py, pz, src_p, dst_p)

    mu = (jnp.arange(RBF_PAD, dtype=f32) * STEP).reshape(1, RBF_PAD)
    mu = mu.at[0, N_RBF:].set(1e6)

    def _padf1(wb):
        w, b = wb
        return jnp.zeros((RBF_PAD, NF), f32).at[:N_RBF].set(w), b.reshape(1, NF)

    f1a, b1a = _padf1(inter[0]['filt1'])
    f2a, b2a = inter[0]['filt2'][0], inter[0]['filt2'][1].reshape(1, NF)
    f1b, b1b = _padf1(inter[1]['filt1'])
    f2b, b2b = inter[1]['filt2'][0], inter[1]['filt2'][1].reshape(1, NF)

    w10, w11, w20, w21 = pl.pallas_call(
        _filter_body,
        grid=(C_GRID,),
        in_specs=[pl.BlockSpec((C_BLK, 1), lambda i: (i, 0)),
                  _full((1, RBF_PAD)),
                  _full((RBF_PAD, NF)), _full((1, NF)),
                  _full((NF, NF)), _full((1, NF)),
                  _full((RBF_PAD, NF)), _full((1, NF)),
                  _full((NF, NF)), _full((1, NF))],
        out_specs=[pl.BlockSpec((C_BLK, HALF), lambda i: (i, 0))] * 4,
        out_shape=[jax.ShapeDtypeStruct((E_PAD, HALF), f32)] * 4,
    )(d2.reshape(E_PAD, 1), mu, f1a, b1a, f2a, b2a, f1b, b1b, f2b, b2b)

    l1w, l1b = inter[0]['lin1'][0], inter[0]['lin1'][1].reshape(1, NF)
    s1w, s1b = p['solv1'][0], p['solv1'][1].reshape(1, 64)
    s2w, s2b = p['solv2'][0], p['solv2'][1].reshape(1, 32)

    h0, m10, m11, s = pl.pallas_call(
        _embed_body,
        grid=(N_GRID,),
        in_specs=[pl.BlockSpec((1, 1, R_BLK), lambda i: (i, 0, 0)),
                  _full((1, 1, B)),
                  _full((100, NF)), _full((4, 64)),
                  _full((64, 64)), _full((1, 64)),
                  _full((64, 32)), _full((1, 32)),
                  _full((NF, NF)), _full((1, NF))],
        out_specs=[pl.BlockSpec((R_BLK, NF), lambda i: (i, 0)),
                   pl.BlockSpec((R_BLK, HALF), lambda i: (i, 0)),
                   pl.BlockSpec((R_BLK, HALF), lambda i: (i, 0)),
                   _full((B, 32))],
        out_shape=[jax.ShapeDtypeStruct((N_NODES, NF), f32),
                   jax.ShapeDtypeStruct((N_NODES, HALF), f32),
                   jax.ShapeDtypeStruct((N_NODES, HALF), f32),
                   jax.ShapeDtypeStruct((B, 32), f32)],
    )(Z.reshape(N_GRID, 1, R_BLK), solvent.reshape(1, 1, B),
      p['emb_z'], p['emb_solv'], s1w, s1b, s2w, s2b, l1w, l1b)

    src2 = src_p.reshape(E_PAD // MSG_BLK, MSG_BLK)
    dst2 = dst_s.reshape(E_PAD // MSG_BLK, MSG_BLK)
    zrows = jnp.zeros((ROWS_PER_TILE, HALF), f32)
    v10, v11 = _msg_call()(src2, dst2, zrows, m10, m11, w10, w11)

    m1w, m1b = inter[0]['mlp1'][0], inter[0]['mlp1'][1].reshape(1, NF)
    m2w, m2b = inter[0]['mlp2'][0], inter[0]['mlp2'][1].reshape(1, NF)
    l2w, l2b = inter[1]['lin1'][0], inter[1]['lin1'][1].reshape(1, NF)

    h1, m20, m21 = pl.pallas_call(
        _update1_body,
        grid=(N_GRID,),
        in_specs=[pl.BlockSpec((R_BLK, NF), lambda i: (i, 0)),
                  pl.BlockSpec((R_BLK, HALF), lambda i: (i, 0)),
                  pl.BlockSpec((R_BLK, HALF), lambda i: (i, 0)),
                  _full((NF, NF)), _full((1, NF)),
                  _full((NF, NF)), _full((1, NF)),
                  _full((NF, NF)), _full((1, NF))],
        out_specs=[pl.BlockSpec((R_BLK, NF), lambda i: (i, 0)),
                   pl.BlockSpec((R_BLK, HALF), lambda i: (i, 0)),
                   pl.BlockSpec((R_BLK, HALF), lambda i: (i, 0))],
        out_shape=[jax.ShapeDtypeStruct((N_NODES, NF), f32),
                   jax.ShapeDtypeStruct((N_NODES, HALF), f32),
                   jax.ShapeDtypeStruct((N_NODES, HALF), f32)],
    )(h0, v10, v11, m1w, m1b, m2w, m2b, l2w, l2b)

    v20, v21 = _msg_call()(src2, dst2, zrows, m20, m21, w20, w21)

    n1w, n1b = inter[1]['mlp1'][0], inter[1]['mlp1'][1].reshape(1, NF)
    n2w, n2b = inter[1]['mlp2'][0], inter[1]['mlp2'][1].reshape(1, NF)
    p1w, p1b = p['post1'][0], p['post1'][1].reshape(1, NF)
    p2w, p2b = p['post2'][0], p['post2'][1].reshape(1, 64)
    paw, pab = p['p2a'][0], p['p2a'][1].reshape(1, 128)
    pbw, pbb = p['p2b'][0], p['p2b'][1].reshape(1, 32)
    pcw, pcb = p['p2c'][0], p['p2c'][1].reshape(1, 1)

    out = pl.pallas_call(
        _final_body,
        grid=(N_GRID,),
        in_specs=[pl.BlockSpec((R_BLK, NF), lambda i: (i, 0)),
                  pl.BlockSpec((R_BLK, HALF), lambda i: (i, 0)),
                  pl.BlockSpec((R_BLK, HALF), lambda i: (i, 0)),
                  pl.BlockSpec((1, 1, R_BLK), lambda i: (i, 0, 0)),
                  _full((B, 32)),
                  _full((NF, NF)), _full((1, NF)),
                  _full((NF, NF)), _full((1, NF)),
                  _full((NF, NF)), _full((1, NF)),
                  _full((NF, 64)), _full((1, 64)),
                  _full((96, 128)), _full((1, 128)),
                  _full((128, 32)), _full((1, 32)),
                  _full((32, 1)), _full((1, 1))],
        out_specs=_full((B, 1)),
        out_shape=jax.ShapeDtypeStruct((B, 1), f32),
        scratch_shapes=[pltpu.VMEM((B, NF), f32), pltpu.VMEM((B, 1), f32)],
    )(h1, v20, v21, batch.reshape(N_GRID, 1, R_BLK), s,
      n1w, n1b, n2w, n2b, p1w, p1b, p2w, p2b, paw, pab, pbw, pbb, pcw, pcb)

    return out


# speedup vs baseline: 1.0295x; 1.0295x over previous
import functools

import jax
import jax.numpy as jnp
from jax import lax
from jax.experimental import pallas as pl
from jax.experimental.pallas import tpu as pltpu
from jax.experimental.pallas import tpu_sc as plsc

N_NODES = 50000
NF = 64
HALF = 32
B = 512
N_RBF = 51
RBF_PAD = 64
GAMMA = 10.0
STEP = 0.1

E_PAD = 802816
EDGES_PER_TILE32 = E_PAD // 32
EDGES_PER_TILE16 = E_PAD // 16
MSG_BLK = 128
MSG_NBLK = EDGES_PER_TILE16 // MSG_BLK
DIST_CHUNK = 3136
DIST_NCHUNK = EDGES_PER_TILE32 // DIST_CHUNK
ACC_ROWS = 50048
DUMP_ROW = N_NODES
ROWS_PER_TILE = ACC_ROWS // 16

R_BLK = 1000
N_GRID = N_NODES // R_BLK
C_BLK = 2048
C_GRID = E_PAD // C_BLK


def _ssp(v):
    return (jnp.maximum(v, 0.0) + jnp.log(jnp.exp(-jnp.abs(v)) + 1.0)
            - 0.6931471805599453)


def _dist_body(px, py, pz, src, dst, d2_out, posbuf, sidx, didx, d2buf, dummy):
    del dummy
    wid = lax.axis_index("s") * 2 + lax.axis_index("c")
    base = wid * EDGES_PER_TILE32

    def _zero(i, _):
        d2buf[pl.ds(i * 16, 16)] = jnp.zeros((16,), jnp.float32)
        return 0
    lax.fori_loop(0, EDGES_PER_TILE32 // 16, _zero, 0)

    for coord in (px, py, pz):
        pltpu.sync_copy(coord, posbuf)

        def _chunk(c, _):
            off = base + c * DIST_CHUNK
            pltpu.sync_copy(src.at[pl.ds(off, DIST_CHUNK)], sidx)
            pltpu.sync_copy(dst.at[pl.ds(off, DIST_CHUNK)], didx)

            def _vec(i, _):
                s16 = sidx[pl.ds(i * 16, 16)]
                d16 = didx[pl.ds(i * 16, 16)]
                a = plsc.load_gather(posbuf, [s16])
                b = plsc.load_gather(posbuf, [d16])
                diff = a - b
                sl = pl.ds(c * DIST_CHUNK + i * 16, 16)
                d2buf[sl] = d2buf[sl] + diff * diff
                return 0
            lax.fori_loop(0, DIST_CHUNK // 16, _vec, 0)
            return 0
        lax.fori_loop(0, DIST_NCHUNK, _chunk, 0)

    pltpu.sync_copy(d2buf, d2_out.at[pl.ds(base, EDGES_PER_TILE32)])


GROUP = 7
PAIR_BLKS = 2 * GROUP
MSG_NPAIR = MSG_NBLK // PAIR_BLKS


def _msg_body(src2, dst2, zeros, m0, m1, w0, w1, v0, v1,
              acc, sidx, didx, mb0, mb1, wb0, wb1, xb0, xb1,
              sg0, sg1, sw0, sw1, ss0, ss1, si0, si1):
    cid = lax.axis_index("c")
    tid = lax.axis_index("s")

    pltpu.sync_copy(zeros, acc.at[pl.ds(tid * ROWS_PER_TILE, ROWS_PER_TILE)])
    plsc.subcore_barrier()

    def _edges(m, w, vout):
        mb = (mb0, mb1)
        wb = (wb0, wb1)
        xb = (xb0, xb1)
        sg = (sg0, sg1)
        sw = (sw0, sw1)
        ss = (ss0, ss1)
        si = (si0, si1)
        tbase128 = tid * (EDGES_PER_TILE16 // MSG_BLK)

        def _issue_idx(pair, p):
            row0 = tbase128 + (2 * pair + p) * GROUP
            pltpu.async_copy(src2.at[pl.ds(row0, GROUP)], sidx.at[p], si[p])
            pltpu.async_copy(dst2.at[pl.ds(row0, GROUP)], didx.at[p], si[p])

        def _drain_idx(p):
            for _ in range(2):
                pltpu.make_async_copy(
                    src2.at[pl.ds(0, GROUP)], sidx.at[p], si[p]).wait()

        _issue_idx(0, 0)
        _issue_idx(0, 1)

        def _pair(g2, _):
            row0 = tbase128 + g2 * PAIR_BLKS
            _drain_idx(0)
            _drain_idx(1)

            def _start(k):
                b = k & 1
                p, r = divmod(k, GROUP)
                hm = pltpu.async_copy(m.at[sidx.at[p, r]], mb[b], sg[b])
                hw = pltpu.async_copy(
                    w.at[pl.ds((row0 + k) * MSG_BLK, MSG_BLK)], wb[b], sw[b])
                return hm, hw

            pend = {0: _start(0), 1: _start(1)}
            pend_s = {}
            for k in range(PAIR_BLKS):
                b = k & 1
                p, r = divmod(k, GROUP)
                hm, hw = pend.pop(k)
                hm.wait()
                hw.wait()
                if k - 2 in pend_s:
                    pend_s.pop(k - 2).wait()

                def _mul(q, _):
                    for rr in range(4):
                        row = q * 4 + rr
                        lo = pl.ds(0, 16)
                        hi = pl.ds(16, 16)
                        xb[b][row, lo] = mb[b][row, lo] * wb[b][row, lo]
                        xb[b][row, hi] = mb[b][row, hi] * wb[b][row, hi]
                    return 0
                lax.fori_loop(0, MSG_BLK // 4, _mul, 0)

                pend_s[k] = pltpu.async_copy(
                    xb[b], acc.at[didx.at[p, r]], ss[b], add=True)
                if k + 2 < PAIR_BLKS:
                    pend[k + 2] = _start(k + 2)
                if k == GROUP + 2:
                    @pl.when(g2 + 1 < MSG_NPAIR)
                    def _():
                        _issue_idx(g2 + 1, 0)
            for k in (PAIR_BLKS - 2, PAIR_BLKS - 1):
                pend_s.pop(k).wait()

            @pl.when(g2 + 1 < MSG_NPAIR)
            def _():
                _issue_idx(g2 + 1, 1)
            return 0
        lax.fori_loop(0, MSG_NPAIR, _pair, 0)
        plsc.subcore_barrier()
        rows = pl.ds(tid * ROWS_PER_TILE, ROWS_PER_TILE)
        pltpu.sync_copy(acc.at[rows], vout.at[rows])

    @pl.when(cid == 0)
    def _():
        _edges(m0, w0, v0)

    @pl.when(cid == 1)
    def _():
        _edges(m1, w1, v1)


def _embed_body(z_ref, solv_ref, embz_ref, embs_ref, s1w, s1b, s2w, s2b,
                l1w, l1b, h_ref, m0_ref, m1_ref, s_ref):
    z = z_ref[0, 0, :].reshape(R_BLK, 1)
    zoh = (z == lax.broadcasted_iota(jnp.int32, (R_BLK, 100), 1)).astype(jnp.float32)
    h = jnp.dot(zoh, embz_ref[...], preferred_element_type=jnp.float32,
                precision=lax.Precision.HIGHEST)
    h_ref[...] = h
    m = jnp.dot(h, l1w[...], preferred_element_type=jnp.float32) + l1b[...]
    m0_ref[...] = m[:, :HALF]
    m1_ref[...] = m[:, HALF:]

    sv = solv_ref[0, 0, :].reshape(B, 1)
    soh = (sv == lax.broadcasted_iota(jnp.int32, (B, 4), 1)).astype(jnp.float32)
    es = jnp.dot(soh, embs_ref[...], preferred_element_type=jnp.float32,
                 precision=lax.Precision.HIGHEST)
    t = _ssp(jnp.dot(es, s1w[...], preferred_element_type=jnp.float32) + s1b[...])
    s_ref[...] = jnp.dot(t, s2w[...], preferred_element_type=jnp.float32) + s2b[...]


def _filter_body(d2_ref, mu_ref, f1a, b1a, f2a, b2a, f1b, b1b, f2b, b2b,
                 w10_ref, w11_ref, w20_ref, w21_ref):
    d = jnp.sqrt(d2_ref[...] + 1e-12)
    delta = d - mu_ref[...]
    rbf = jnp.exp(-GAMMA * delta * delta)
    t = _ssp(jnp.dot(rbf, f1a[...], preferred_element_type=jnp.float32) + b1a[...])
    w = jnp.dot(t, f2a[...], preferred_element_type=jnp.float32) + b2a[...]
    w10_ref[...] = w[:, :HALF]
    w11_ref[...] = w[:, HALF:]
    t = _ssp(jnp.dot(rbf, f1b[...], preferred_element_type=jnp.float32) + b1b[...])
    w = jnp.dot(t, f2b[...], preferred_element_type=jnp.float32) + b2b[...]
    w20_ref[...] = w[:, :HALF]
    w21_ref[...] = w[:, HALF:]


def _update1_body(h_ref, v0_ref, v1_ref, m1w, m1b, m2w, m2b, l2w, l2b,
                  h1_ref, m0_ref, m1_ref):
    v = jnp.concatenate([v0_ref[...], v1_ref[...]], axis=1)
    t = _ssp(jnp.dot(v, m1w[...], preferred_element_type=jnp.float32) + m1b[...])
    t = jnp.dot(t, m2w[...], preferred_element_type=jnp.float32) + m2b[...]
    h1 = h_ref[...] + t
    h1_ref[...] = h1
    m = jnp.dot(h1, l2w[...], preferred_element_type=jnp.float32) + l2b[...]
    m0_ref[...] = m[:, :HALF]
    m1_ref[...] = m[:, HALF:]


def _final_body(h_ref, v0_ref, v1_ref, batch_ref, s_ref,
                m1w, m1b, m2w, m2b, p1w, p1b, p2w, p2b,
                paw, pab, pbw, pbb, pcw, pcb,
                out_ref, acc, cnt):
    i = pl.program_id(0)
    v = jnp.concatenate([v0_ref[...], v1_ref[...]], axis=1)
    t = _ssp(jnp.dot(v, m1w[...], preferred_element_type=jnp.float32) + m1b[...])
    t = jnp.dot(t, m2w[...], preferred_element_type=jnp.float32) + m2b[...]
    h = h_ref[...] + t
    t = _ssp(jnp.dot(h, p1w[...], preferred_element_type=jnp.float32) + p1b[...])
    ht = jnp.dot(t, p2w[...], preferred_element_type=jnp.float32) + p2b[...]

    bv = batch_ref[0, 0, :].reshape(R_BLK, 1)
    oh = (bv == lax.broadcasted_iota(jnp.int32, (R_BLK, B), 1)).astype(jnp.float32)

    @pl.when(i == 0)
    def _():
        acc[...] = jnp.zeros((B, NF), jnp.float32)
        cnt[...] = jnp.zeros((B, 1), jnp.float32)

    dn = (((0,), (0,)), ((), ()))
    acc[...] = acc[...] + lax.dot_general(oh, ht, dn,
                                          preferred_element_type=jnp.float32,
                                          precision=lax.Precision.HIGHEST)
    cnt[...] = cnt[...] + lax.dot_general(
        oh, jnp.ones((R_BLK, 1), jnp.float32), dn,
        preferred_element_type=jnp.float32)

    @pl.when(i == N_GRID - 1)
    def _():
        gmean = acc[...] / jnp.maximum(cnt[...], 1.0)
        cat = jnp.concatenate([gmean, s_ref[...]], axis=1)
        o = _ssp(jnp.dot(cat, paw[...], preferred_element_type=jnp.float32) + pab[...])
        o = _ssp(jnp.dot(o, pbw[...], preferred_element_type=jnp.float32) + pbb[...])
        out_ref[...] = jnp.dot(o, pcw[...], preferred_element_type=jnp.float32) + pcb[...]


_MESH = plsc.VectorSubcoreMesh(core_axis_name="c", subcore_axis_name="s")
_SC_PARAMS = pltpu.CompilerParams(needs_layout_passes=False,
                                  use_tc_tiling_on_sc=False)

_dist_call = functools.partial(
    pl.kernel, _dist_body,
    out_type=jax.ShapeDtypeStruct((E_PAD,), jnp.float32),
    mesh=_MESH,
    scratch_types=[
        pltpu.VMEM((N_NODES,), jnp.float32),
        pltpu.VMEM((DIST_CHUNK,), jnp.int32),
        pltpu.VMEM((DIST_CHUNK,), jnp.int32),
        pltpu.VMEM((EDGES_PER_TILE32,), jnp.float32),
        pltpu.SemaphoreType.DMA,
    ],
    compiler_params=_SC_PARAMS,
)

_msg_call = functools.partial(
    pl.kernel, _msg_body,
    out_type=(jax.ShapeDtypeStruct((ACC_ROWS, HALF), jnp.float32),
              jax.ShapeDtypeStruct((ACC_ROWS, HALF), jnp.float32)),
    mesh=_MESH,
    scratch_types=[
        pltpu.VMEM_SHARED((ACC_ROWS, HALF), jnp.float32),
        pltpu.VMEM((2, GROUP, MSG_BLK), jnp.int32),
        pltpu.VMEM((2, GROUP, MSG_BLK), jnp.int32),
        pltpu.VMEM((MSG_BLK, HALF), jnp.float32),
        pltpu.VMEM((MSG_BLK, HALF), jnp.float32),
        pltpu.VMEM((MSG_BLK, HALF), jnp.float32),
        pltpu.VMEM((MSG_BLK, HALF), jnp.float32),
        pltpu.VMEM((MSG_BLK, HALF), jnp.float32),
        pltpu.VMEM((MSG_BLK, HALF), jnp.float32),
        pltpu.SemaphoreType.DMA,
        pltpu.SemaphoreType.DMA,
        pltpu.SemaphoreType.DMA,
        pltpu.SemaphoreType.DMA,
        pltpu.SemaphoreType.DMA,
        pltpu.SemaphoreType.DMA,
        pltpu.SemaphoreType.DMA,
        pltpu.SemaphoreType.DMA,
    ],
    compiler_params=_SC_PARAMS,
)


def _full(shape):
    return pl.BlockSpec(shape, lambda i: tuple(0 for _ in shape))


def kernel(x, pos, edge_index, Z, batch, solvent, nuc_index, params):
    del x, nuc_index
    f32 = jnp.float32
    p = params
    inter = p['interactions']

    src = edge_index[0]
    dst = edge_index[1]
    n_e = src.shape[0]
    pad = E_PAD - n_e
    src_p = jnp.concatenate([src, jnp.zeros((pad,), jnp.int32)])
    dst_p = jnp.concatenate([dst, jnp.zeros((pad,), jnp.int32)])
    dst_s = jnp.concatenate([dst, jnp.full((pad,), DUMP_ROW, jnp.int32)])

    px = pos[:, 0]
    py = pos[:, 1]
    pz = pos[:, 2]

    d2 = _dist_call()(px, py, pz, src_p, dst_p)

    mu = (jnp.arange(RBF_PAD, dtype=f32) * STEP).reshape(1, RBF_PAD)
    mu = mu.at[0, N_RBF:].set(1e6)

    def _padf1(wb):
        w, b = wb
        return jnp.zeros((RBF_PAD, NF), f32).at[:N_RBF].set(w), b.reshape(1, NF)

    f1a, b1a = _padf1(inter[0]['filt1'])
    f2a, b2a = inter[0]['filt2'][0], inter[0]['filt2'][1].reshape(1, NF)
    f1b, b1b = _padf1(inter[1]['filt1'])
    f2b, b2b = inter[1]['filt2'][0], inter[1]['filt2'][1].reshape(1, NF)

    w10, w11, w20, w21 = pl.pallas_call(
        _filter_body,
        grid=(C_GRID,),
        in_specs=[pl.BlockSpec((C_BLK, 1), lambda i: (i, 0)),
                  _full((1, RBF_PAD)),
                  _full((RBF_PAD, NF)), _full((1, NF)),
                  _full((NF, NF)), _full((1, NF)),
                  _full((RBF_PAD, NF)), _full((1, NF)),
                  _full((NF, NF)), _full((1, NF))],
        out_specs=[pl.BlockSpec((C_BLK, HALF), lambda i: (i, 0))] * 4,
        out_shape=[jax.ShapeDtypeStruct((E_PAD, HALF), f32)] * 4,
    )(d2.reshape(E_PAD, 1), mu, f1a, b1a, f2a, b2a, f1b, b1b, f2b, b2b)

    l1w, l1b = inter[0]['lin1'][0], inter[0]['lin1'][1].reshape(1, NF)
    s1w, s1b = p['solv1'][0], p['solv1'][1].reshape(1, 64)
    s2w, s2b = p['solv2'][0], p['solv2'][1].reshape(1, 32)

    h0, m10, m11, s = pl.pallas_call(
        _embed_body,
        grid=(N_GRID,),
        in_specs=[pl.BlockSpec((1, 1, R_BLK), lambda i: (i, 0, 0)),
                  _full((1, 1, B)),
                  _full((100, NF)), _full((4, 64)),
                  _full((64, 64)), _full((1, 64)),
                  _full((64, 32)), _full((1, 32)),
                  _full((NF, NF)), _full((1, NF))],
        out_specs=[pl.BlockSpec((R_BLK, NF), lambda i: (i, 0)),
                   pl.BlockSpec((R_BLK, HALF), lambda i: (i, 0)),
                   pl.BlockSpec((R_BLK, HALF), lambda i: (i, 0)),
                   _full((B, 32))],
        out_shape=[jax.ShapeDtypeStruct((N_NODES, NF), f32),
                   jax.ShapeDtypeStruct((N_NODES, HALF), f32),
                   jax.ShapeDtypeStruct((N_NODES, HALF), f32),
                   jax.ShapeDtypeStruct((B, 32), f32)],
    )(Z.reshape(N_GRID, 1, R_BLK), solvent.reshape(1, 1, B),
      p['emb_z'], p['emb_solv'], s1w, s1b, s2w, s2b, l1w, l1b)

    src2 = src_p.reshape(E_PAD // MSG_BLK, MSG_BLK)
    dst2 = dst_s.reshape(E_PAD // MSG_BLK, MSG_BLK)
    zrows = jnp.zeros((ROWS_PER_TILE, HALF), f32)
    v10, v11 = _msg_call()(src2, dst2, zrows, m10, m11, w10, w11)

    m1w, m1b = inter[0]['mlp1'][0], inter[0]['mlp1'][1].reshape(1, NF)
    m2w, m2b = inter[0]['mlp2'][0], inter[0]['mlp2'][1].reshape(1, NF)
    l2w, l2b = inter[1]['lin1'][0], inter[1]['lin1'][1].reshape(1, NF)

    h1, m20, m21 = pl.pallas_call(
        _update1_body,
        grid=(N_GRID,),
        in_specs=[pl.BlockSpec((R_BLK, NF), lambda i: (i, 0)),
                  pl.BlockSpec((R_BLK, HALF), lambda i: (i, 0)),
                  pl.BlockSpec((R_BLK, HALF), lambda i: (i, 0)),
                  _full((NF, NF)), _full((1, NF)),
                  _full((NF, NF)), _full((1, NF)),
                  _full((NF, NF)), _full((1, NF))],
        out_specs=[pl.BlockSpec((R_BLK, NF), lambda i: (i, 0)),
                   pl.BlockSpec((R_BLK, HALF), lambda i: (i, 0)),
                   pl.BlockSpec((R_BLK, HALF), lambda i: (i, 0))],
        out_shape=[jax.ShapeDtypeStruct((N_NODES, NF), f32),
                   jax.ShapeDtypeStruct((N_NODES, HALF), f32),
                   jax.ShapeDtypeStruct((N_NODES, HALF), f32)],
    )(h0, v10, v11, m1w, m1b, m2w, m2b, l2w, l2b)

    v20, v21 = _msg_call()(src2, dst2, zrows, m20, m21, w20, w21)

    n1w, n1b = inter[1]['mlp1'][0], inter[1]['mlp1'][1].reshape(1, NF)
    n2w, n2b = inter[1]['mlp2'][0], inter[1]['mlp2'][1].reshape(1, NF)
    p1w, p1b = p['post1'][0], p['post1'][1].reshape(1, NF)
    p2w, p2b = p['post2'][0], p['post2'][1].reshape(1, 64)
    paw, pab = p['p2a'][0], p['p2a'][1].reshape(1, 128)
    pbw, pbb = p['p2b'][0], p['p2b'][1].reshape(1, 32)
    pcw, pcb = p['p2c'][0], p['p2c'][1].reshape(1, 1)

    out = pl.pallas_call(
        _final_body,
        grid=(N_GRID,),
        in_specs=[pl.BlockSpec((R_BLK, NF), lambda i: (i, 0)),
                  pl.BlockSpec((R_BLK, HALF), lambda i: (i, 0)),
                  pl.BlockSpec((R_BLK, HALF), lambda i: (i, 0)),
                  pl.BlockSpec((1, 1, R_BLK), lambda i: (i, 0, 0)),
                  _full((B, 32)),
                  _full((NF, NF)), _full((1, NF)),
                  _full((NF, NF)), _full((1, NF)),
                  _full((NF, NF)), _full((1, NF)),
                  _full((NF, 64)), _full((1, 64)),
                  _full((96, 128)), _full((1, 128)),
                  _full((128, 32)), _full((1, 32)),
                  _full((32, 1)), _full((1, 1))],
        out_specs=_full((B, 1)),
        out_shape=jax.ShapeDtypeStruct((B, 1), f32),
        scratch_shapes=[pltpu.VMEM((B, NF), f32), pltpu.VMEM((B, 1), f32)],
    )(h1, v20, v21, batch.reshape(N_GRID, 1, R_BLK), s,
      n1w, n1b, n2w, n2b, p1w, p1b, p2w, p2b, paw, pab, pbw, pbb, pcw, pcb)

    return out
